# baseline (speedup 1.0000x reference)
import numpy as np

import jax
import jax.numpy as jnp
from jax import lax
from jax.experimental import pallas as pl
from jax.experimental.pallas import tpu as pltpu


def _interp_weights_f32(out_size, in_size):
    scale = (in_size - 1) / (out_size - 1) if out_size > 1 else 0.0
    coords = np.arange(out_size, dtype=np.float32) * np.float32(scale)
    lo = coords.astype(np.int64)
    hi = np.minimum(np.ceil(coords), in_size - 1).astype(np.int64)
    frac = coords - lo.astype(np.float32)
    m = np.zeros((out_size, in_size), dtype=np.float32)
    r = np.arange(out_size)
    m[r, lo] += 1.0 - frac
    m[r, hi] += frac
    return m


_B = 32
_NSLOT = 6
_NREAD = 4


def _bilerp_manual_kernel(
    ww_ref, wh_ref, x_hbm, o_hbm, x_vmem, o_vmem, read_sems, write_sems
):
    NCH, H, W = x_vmem.shape
    OH, OW = o_vmem.shape[2], o_vmem.shape[3]
    n_chunks = NCH // _B
    piece = NCH // _NREAD

    tc = pl.program_id(0)
    base = tc * NCH

    reads = []
    for j in range(_NREAD):
        rd = pltpu.make_async_copy(
            x_hbm.at[pl.ds(base + j * piece, piece)],
            x_vmem.at[pl.ds(j * piece, piece)],
            read_sems.at[j],
        )
        rd.start()
        reads.append(rd)

    writes = [None] * _NSLOT
    for k in range(n_chunks):
        slot = k % _NSLOT
        if (k * _B) % piece == 0:
            reads[(k * _B) // piece].wait()
        if writes[slot] is not None:
            writes[slot].wait()
        xb = x_vmem[pl.ds(k * _B, _B)].astype(jnp.bfloat16).reshape(_B * H, W)
        t = jnp.dot(xb, ww_ref[...], preferred_element_type=jnp.float32)
        tb = t.astype(jnp.bfloat16).reshape(_B, H, OW)
        wh_b = jnp.broadcast_to(wh_ref[...], (_B, OH, H))
        o = lax.dot_general(
            wh_b,
            tb,
            dimension_numbers=(((2,), (1,)), ((0,), (0,))),
            preferred_element_type=jnp.float32,
        )
        o_vmem[slot] = o
        wr = pltpu.make_async_copy(
            o_vmem.at[slot],
            o_hbm.at[pl.ds(base + k * _B, _B)],
            write_sems.at[slot],
        )
        wr.start()
        writes[slot] = wr
    for wr in writes:
        if wr is not None:
            wr.wait()


def kernel(x):
    N, C, H, W = x.shape
    OH, OW = 128, 128
    NC = N * C
    NCH = NC // 2
    assert NCH % (_B * _NREAD) == 0 and (NCH // _NREAD) % _B == 0

    wh = jnp.asarray(_interp_weights_f32(OH, H), dtype=jnp.bfloat16)
    wwt = jnp.asarray(
        np.ascontiguousarray(_interp_weights_f32(OW, W).T), dtype=jnp.bfloat16
    )
    x3 = x.reshape(NC, H, W)

    out = pl.pallas_call(
        _bilerp_manual_kernel,
        out_shape=jax.ShapeDtypeStruct((NC, OH, OW), jnp.float32),
        grid=(2,),
        in_specs=[
            pl.BlockSpec((W, OW), lambda i: (0, 0)),
            pl.BlockSpec((OH, H), lambda i: (0, 0)),
            pl.BlockSpec(memory_space=pltpu.MemorySpace.HBM),
        ],
        out_specs=pl.BlockSpec(memory_space=pltpu.MemorySpace.HBM),
        scratch_shapes=[
            pltpu.VMEM((NCH, H, W), jnp.float32),
            pltpu.VMEM((_NSLOT, _B, OH, OW), jnp.float32),
            pltpu.SemaphoreType.DMA((_NREAD,)),
            pltpu.SemaphoreType.DMA((_NSLOT,)),
        ],
        compiler_params=pltpu.CompilerParams(
            dimension_semantics=("parallel",),
        ),
    )(wwt, wh, x3)
    return out.reshape(N, C, OH, OW)


# speedup vs baseline: 1.0028x; 1.0028x over previous
import numpy as np

import jax
import jax.numpy as jnp
from jax import lax
from jax.experimental import pallas as pl
from jax.experimental.pallas import tpu as pltpu


def _interp_weights_f32(out_size, in_size):
    scale = (in_size - 1) / (out_size - 1) if out_size > 1 else 0.0
    coords = np.arange(out_size, dtype=np.float32) * np.float32(scale)
    lo = coords.astype(np.int64)
    hi = np.minimum(np.ceil(coords), in_size - 1).astype(np.int64)
    frac = coords - lo.astype(np.float32)
    m = np.zeros((out_size, in_size), dtype=np.float32)
    r = np.arange(out_size)
    m[r, lo] += 1.0 - frac
    m[r, hi] += frac
    return m


_B = 64
_NSLOT = 4
_NREAD = 4


def _bilerp_manual_kernel(
    ww_ref, wh_ref, x_hbm, o_hbm, x_vmem, o_vmem, read_sems, write_sems
):
    NCH, H, W = x_vmem.shape
    OH, OW = o_vmem.shape[2], o_vmem.shape[3]
    n_chunks = NCH // _B
    piece = NCH // _NREAD

    tc = pl.program_id(0)
    base = tc * NCH

    reads = []
    for j in range(_NREAD):
        rd = pltpu.make_async_copy(
            x_hbm.at[pl.ds(base + j * piece, piece)],
            x_vmem.at[pl.ds(j * piece, piece)],
            read_sems.at[j],
        )
        rd.start()
        reads.append(rd)

    writes = [None] * _NSLOT
    for k in range(n_chunks):
        slot = k % _NSLOT
        if (k * _B) % piece == 0:
            reads[(k * _B) // piece].wait()
        if writes[slot] is not None:
            writes[slot].wait()
        xb = x_vmem[pl.ds(k * _B, _B)].astype(jnp.bfloat16).reshape(_B * H, W)
        t = jnp.dot(xb, ww_ref[...], preferred_element_type=jnp.float32)
        tb = t.astype(jnp.bfloat16).reshape(_B, H, OW)
        wh_b = jnp.broadcast_to(wh_ref[...], (_B, OH, H))
        o = lax.dot_general(
            wh_b,
            tb,
            dimension_numbers=(((2,), (1,)), ((0,), (0,))),
            preferred_element_type=jnp.float32,
        )
        o_vmem[slot] = o
        wr = pltpu.make_async_copy(
            o_vmem.at[slot],
            o_hbm.at[pl.ds(base + k * _B, _B)],
            write_sems.at[slot],
        )
        wr.start()
        writes[slot] = wr
    for wr in writes:
        if wr is not None:
            wr.wait()


def kernel(x):
    N, C, H, W = x.shape
    OH, OW = 128, 128
    NC = N * C
    NCH = NC // 2
    assert NCH % (_B * _NREAD) == 0 and (NCH // _NREAD) % _B == 0

    wh = jnp.asarray(_interp_weights_f32(OH, H), dtype=jnp.bfloat16)
    wwt = jnp.asarray(
        np.ascontiguousarray(_interp_weights_f32(OW, W).T), dtype=jnp.bfloat16
    )
    x3 = x.reshape(NC, H, W)

    out = pl.pallas_call(
        _bilerp_manual_kernel,
        out_shape=jax.ShapeDtypeStruct((NC, OH, OW), jnp.float32),
        grid=(2,),
        in_specs=[
            pl.BlockSpec((W, OW), lambda i: (0, 0)),
            pl.BlockSpec((OH, H), lambda i: (0, 0)),
            pl.BlockSpec(memory_space=pltpu.MemorySpace.HBM),
        ],
        out_specs=pl.BlockSpec(memory_space=pltpu.MemorySpace.HBM),
        scratch_shapes=[
            pltpu.VMEM((NCH, H, W), jnp.float32),
            pltpu.VMEM((_NSLOT, _B, OH, OW), jnp.float32),
            pltpu.SemaphoreType.DMA((_NREAD,)),
            pltpu.SemaphoreType.DMA((_NSLOT,)),
        ],
        compiler_params=pltpu.CompilerParams(
            dimension_semantics=("parallel",),
        ),
    )(wwt, wh, x3)
    return out.reshape(N, C, OH, OW)
